# baseline (speedup 1.0000x reference)
import functools

import jax
import jax.numpy as jnp
from jax import lax
from jax.experimental import pallas as pl
from jax.experimental.pallas import tpu as pltpu
from jax.experimental.pallas import tpu_sc as plsc

EMB = 64
LANE = 128
N = 1000000
LB = 8192
NQ = 31
Q = NQ * LB
TOTB = -(-N // LB)
HI = 0xFFFF0000


def _pack(ra, rb):
    ua = lax.bitcast_convert_type(ra, jnp.uint32)
    ub = lax.bitcast_convert_type(rb, jnp.uint32)
    packed = (ua & jnp.uint32(HI)) | lax.shift_right_logical(
        ub, jnp.uint32(16))
    return lax.bitcast_convert_type(packed, jnp.float32)


def _t_body(a_ref, b_ref, c_ref, d_ref, i_ref, o_ref):
    ident = i_ref[...]
    dn = (((0,), (0,)), ((), ()))

    def tr(ref):
        return lax.dot_general(ref[...], ident, dn,
                               preferred_element_type=jnp.float32)

    o_ref[:, 0:EMB] = _pack(tr(a_ref), tr(c_ref))
    o_ref[:, EMB:LANE] = _pack(tr(b_ref), tr(d_ref))


def _retile(tt, ident):
    bmap = [lambda i, k=k: (0, jnp.minimum(i + k * NQ, TOTB - 1))
            for k in range(4)]
    return pl.pallas_call(
        _t_body,
        grid=(NQ,),
        in_specs=[pl.BlockSpec((EMB, LB), bmap[k]) for k in range(4)]
        + [pl.BlockSpec((EMB, EMB), lambda i: (0, 0))],
        out_specs=pl.BlockSpec((LB, LANE), lambda i: (i, 0)),
        out_shape=jax.ShapeDtypeStruct((Q, LANE), jnp.float32),
    )(tt, tt, tt, tt, ident)


def _make_gather(B):
    info = plsc.get_sparse_core_info()
    NC, NS = info.num_cores, info.num_subcores
    NW = NC * NS
    b_per_w = B // NW

    mesh = plsc.VectorSubcoreMesh(core_axis_name="c", subcore_axis_name="s")

    @functools.partial(
        pl.kernel,
        mesh=mesh,
        compiler_params=pltpu.CompilerParams(
            use_tc_tiling_on_sc=True, needs_layout_passes=False),
        out_type=jax.ShapeDtypeStruct((B, LANE), jnp.float32),
        scratch_types=[
            pltpu.VMEM((b_per_w,), jnp.int32),
            pltpu.VMEM((b_per_w, LANE), jnp.float32),
            pltpu.SemaphoreType.DMA,
        ],
    )
    def gather_k(ix_hbm, emb_hbm, out_hbm, idxv, rows, gsem):
        wid = lax.axis_index("s") * NC + lax.axis_index("c")
        base = wid * b_per_w
        lanes = lax.iota(jnp.int32, 16)

        pltpu.sync_copy(ix_hbm.at[pl.ds(base, b_per_w)], idxv)
        for g in range(b_per_w // 16):
            sl = pl.ds(g * 16, 16)
            s = idxv[sl]
            k = (jnp.where(s >= Q, 1, 0) + jnp.where(s >= 2 * Q, 1, 0)
                 + jnp.where(s >= 3 * Q, 1, 0))
            idxv[sl] = s - k * Q

        def body(g, _):
            vec = idxv[pl.ds(g * 16, 16)]
            for l in range(16):
                s = jnp.sum(jnp.where(lanes == l, vec, 0))
                pltpu.async_copy(emb_hbm.at[s], rows.at[g * 16 + l], gsem)
            return 0

        lax.fori_loop(0, b_per_w // 16, body, 0)
        pltpu.make_async_copy(out_hbm.at[pl.ds(base, b_per_w)],
                              rows, gsem).wait()
        pltpu.sync_copy(rows, out_hbm.at[pl.ds(base, b_per_w)])

    return gather_k


def _unpack_select(x2_ref, xi_ref):
    r = xi_ref[...]
    k = (jnp.where(r >= Q, 1, 0) + jnp.where(r >= 2 * Q, 1, 0)
         + jnp.where(r >= 3 * Q, 1, 0))
    x = jnp.where((k & 1) == 1, x2_ref[:, EMB:LANE], x2_ref[:, 0:EMB])
    w = lax.bitcast_convert_type(x, jnp.uint32)
    hi = lax.bitcast_convert_type(w & jnp.uint32(HI), jnp.float32)
    lo = lax.bitcast_convert_type(lax.shift_left(w, jnp.uint32(16)),
                                  jnp.float32)
    return jnp.where(k >= 2, lo, hi)


def _mlp_body(u2_ref, v2_ref, ui_ref, vi_ref, w1a_ref, w1b_ref, b1_ref,
              w2_ref, b2_ref, o_ref):
    u = jnp.maximum(_unpack_select(u2_ref, ui_ref), 0.0)
    v = jnp.maximum(_unpack_select(v2_ref, vi_ref), 0.0)
    h = jnp.dot(u, w1a_ref[...], preferred_element_type=jnp.float32)
    h = h + jnp.dot(v, w1b_ref[...], preferred_element_type=jnp.float32)
    h = jnp.maximum(h + b1_ref[...], 0.0)
    o_ref[...] = jnp.sum(h * w2_ref[...], axis=1, keepdims=True) + b2_ref[...]


def kernel(u, v, user_emb, item_emb, W1, b1, W2, b2):
    B = u.shape[0]
    u32 = u.astype(jnp.int32)
    v32 = v.astype(jnp.int32)
    ident = jnp.eye(EMB, dtype=jnp.float32)
    gather_k = _make_gather(B)
    TU = _retile(user_emb.T, ident)
    U2 = gather_k(u32, TU)
    TV = _retile(item_emb.T, ident)
    V2 = gather_k(v32, TV)

    BLK = 4096
    grid = (B // BLK,)
    out = pl.pallas_call(
        _mlp_body,
        grid=grid,
        in_specs=[
            pl.BlockSpec((BLK, LANE), lambda i: (i, 0)),
            pl.BlockSpec((BLK, LANE), lambda i: (i, 0)),
            pl.BlockSpec((BLK, 1), lambda i: (i, 0)),
            pl.BlockSpec((BLK, 1), lambda i: (i, 0)),
            pl.BlockSpec((EMB, EMB), lambda i: (0, 0)),
            pl.BlockSpec((EMB, EMB), lambda i: (0, 0)),
            pl.BlockSpec((1, EMB), lambda i: (0, 0)),
            pl.BlockSpec((1, EMB), lambda i: (0, 0)),
            pl.BlockSpec((1, 1), lambda i: (0, 0)),
        ],
        out_specs=pl.BlockSpec((BLK, 1), lambda i: (i, 0)),
        out_shape=jax.ShapeDtypeStruct((B, 1), jnp.float32),
    )(U2, V2, u32.reshape(B, 1), v32.reshape(B, 1), W1[:EMB], W1[EMB:],
      b1.reshape(1, EMB), W2.reshape(1, EMB), b2.reshape(1, 1))
    return out


# speedup vs baseline: 1.2499x; 1.2499x over previous
import functools

import jax
import jax.numpy as jnp
from jax import lax
from jax.experimental import pallas as pl
from jax.experimental.pallas import tpu as pltpu
from jax.experimental.pallas import tpu_sc as plsc

EMB = 64
LANE = 128
N = 1000000
LB = 8192
NQ = 31
Q = NQ * LB
TOTB = -(-N // LB)
HI = 0xFFFF0000


def _pack(ra, rb):
    ua = lax.bitcast_convert_type(ra, jnp.uint32)
    ub = lax.bitcast_convert_type(rb, jnp.uint32)
    packed = (ua & jnp.uint32(HI)) | lax.shift_right_logical(
        ub, jnp.uint32(16))
    return lax.bitcast_convert_type(packed, jnp.float32)


def _t_body(a_ref, b_ref, c_ref, d_ref, i_ref, o_ref):
    ident = i_ref[...].astype(jnp.bfloat16)
    dn = (((0,), (0,)), ((), ()))

    def tr(ref):
        return lax.dot_general(ref[...].astype(jnp.bfloat16), ident, dn,
                               preferred_element_type=jnp.float32)

    o_ref[:, 0:EMB] = _pack(tr(a_ref), tr(c_ref))
    o_ref[:, EMB:LANE] = _pack(tr(b_ref), tr(d_ref))


def _retile(tt, ident):
    bmap = [lambda i, k=k: (0, jnp.minimum(i + k * NQ, TOTB - 1))
            for k in range(4)]
    return pl.pallas_call(
        _t_body,
        grid=(NQ,),
        in_specs=[pl.BlockSpec((EMB, LB), bmap[k]) for k in range(4)]
        + [pl.BlockSpec((EMB, EMB), lambda i: (0, 0))],
        out_specs=pl.BlockSpec((LB, LANE), lambda i: (i, 0)),
        out_shape=jax.ShapeDtypeStruct((Q, LANE), jnp.float32),
    )(tt, tt, tt, tt, ident)


def _make_gather(B):
    info = plsc.get_sparse_core_info()
    NC, NS = info.num_cores, info.num_subcores
    NW = NC * NS
    b_per_w = B // NW

    mesh = plsc.VectorSubcoreMesh(core_axis_name="c", subcore_axis_name="s")

    @functools.partial(
        pl.kernel,
        mesh=mesh,
        compiler_params=pltpu.CompilerParams(
            use_tc_tiling_on_sc=True, needs_layout_passes=False),
        out_type=jax.ShapeDtypeStruct((B, LANE), jnp.float32),
        scratch_types=[
            pltpu.VMEM((b_per_w,), jnp.int32),
            pltpu.VMEM((b_per_w, LANE), jnp.float32),
            pltpu.SemaphoreType.DMA,
        ],
    )
    def gather_k(ix_hbm, emb_hbm, out_hbm, idxv, rows, gsem):
        wid = lax.axis_index("s") * NC + lax.axis_index("c")
        base = wid * b_per_w
        lanes = lax.iota(jnp.int32, 16)

        pltpu.sync_copy(ix_hbm.at[pl.ds(base, b_per_w)], idxv)
        for g in range(b_per_w // 16):
            sl = pl.ds(g * 16, 16)
            s = idxv[sl]
            k = (jnp.where(s >= Q, 1, 0) + jnp.where(s >= 2 * Q, 1, 0)
                 + jnp.where(s >= 3 * Q, 1, 0))
            idxv[sl] = s - k * Q

        def body(g, _):
            vec = idxv[pl.ds(g * 16, 16)]
            for l in range(16):
                s = jnp.sum(jnp.where(lanes == l, vec, 0))
                pltpu.async_copy(emb_hbm.at[s], rows.at[g * 16 + l], gsem)
            return 0

        lax.fori_loop(0, b_per_w // 16, body, 0)
        pltpu.make_async_copy(out_hbm.at[pl.ds(base, b_per_w)],
                              rows, gsem).wait()
        pltpu.sync_copy(rows, out_hbm.at[pl.ds(base, b_per_w)])

    return gather_k


def _unpack_select(x2_ref, xi_ref):
    r = xi_ref[...]
    k = (jnp.where(r >= Q, 1, 0) + jnp.where(r >= 2 * Q, 1, 0)
         + jnp.where(r >= 3 * Q, 1, 0))
    x = jnp.where((k & 1) == 1, x2_ref[:, EMB:LANE], x2_ref[:, 0:EMB])
    w = lax.bitcast_convert_type(x, jnp.uint32)
    hi = lax.bitcast_convert_type(w & jnp.uint32(HI), jnp.float32)
    lo = lax.bitcast_convert_type(lax.shift_left(w, jnp.uint32(16)),
                                  jnp.float32)
    return jnp.where(k >= 2, lo, hi)


def _mlp_body(u2_ref, v2_ref, ui_ref, vi_ref, w1a_ref, w1b_ref, b1_ref,
              w2_ref, b2_ref, o_ref):
    u = jnp.maximum(_unpack_select(u2_ref, ui_ref), 0.0)
    v = jnp.maximum(_unpack_select(v2_ref, vi_ref), 0.0)
    h = jnp.dot(u, w1a_ref[...], preferred_element_type=jnp.float32)
    h = h + jnp.dot(v, w1b_ref[...], preferred_element_type=jnp.float32)
    h = jnp.maximum(h + b1_ref[...], 0.0)
    o_ref[...] = jnp.sum(h * w2_ref[...], axis=1, keepdims=True) + b2_ref[...]


def kernel(u, v, user_emb, item_emb, W1, b1, W2, b2):
    B = u.shape[0]
    u32 = u.astype(jnp.int32)
    v32 = v.astype(jnp.int32)
    ident = jnp.eye(EMB, dtype=jnp.float32)
    gather_k = _make_gather(B)
    TU = _retile(user_emb.T, ident)
    U2 = gather_k(u32, TU)
    TV = _retile(item_emb.T, ident)
    V2 = gather_k(v32, TV)

    BLK = 2048
    grid = (B // BLK,)
    out = pl.pallas_call(
        _mlp_body,
        grid=grid,
        in_specs=[
            pl.BlockSpec((BLK, LANE), lambda i: (i, 0)),
            pl.BlockSpec((BLK, LANE), lambda i: (i, 0)),
            pl.BlockSpec((BLK, 1), lambda i: (i, 0)),
            pl.BlockSpec((BLK, 1), lambda i: (i, 0)),
            pl.BlockSpec((EMB, EMB), lambda i: (0, 0)),
            pl.BlockSpec((EMB, EMB), lambda i: (0, 0)),
            pl.BlockSpec((1, EMB), lambda i: (0, 0)),
            pl.BlockSpec((1, EMB), lambda i: (0, 0)),
            pl.BlockSpec((1, 1), lambda i: (0, 0)),
        ],
        out_specs=pl.BlockSpec((BLK, 1), lambda i: (i, 0)),
        out_shape=jax.ShapeDtypeStruct((B, 1), jnp.float32),
    )(U2, V2, u32.reshape(B, 1), v32.reshape(B, 1), W1[:EMB], W1[EMB:],
      b1.reshape(1, EMB), W2.reshape(1, EMB), b2.reshape(1, 1))
    return out
